# baseline (speedup 1.0000x reference)
import functools

import jax
import jax.numpy as jnp
from jax.experimental import pallas as pl
from jax.experimental.pallas import tpu as pltpu

T = 32768
D_MODEL = 768
N_EXPERTS = 8
TOP_K = 2

BLOCK_T = 2048
N_SPLIT = 3
D_CHUNK = D_MODEL // N_SPLIT


def _router_kernel(h0_ref, h1_ref, h2_ref, wt_ref, mask_ref, weight_ref, logits_ref):
    wt = wt_ref[...]
    dn = (((1,), (0,)), ((), ()))
    logits = jax.lax.dot_general(
        h0_ref[...], wt[:D_CHUNK], dn, preferred_element_type=jnp.float32
    )
    logits += jax.lax.dot_general(
        h1_ref[...], wt[D_CHUNK : 2 * D_CHUNK], dn, preferred_element_type=jnp.float32
    )
    logits += jax.lax.dot_general(
        h2_ref[...], wt[2 * D_CHUNK :], dn, preferred_element_type=jnp.float32
    )

    lt = logits.T
    logits_ref[...] = lt

    m1 = jnp.max(lt, axis=0, keepdims=True)
    e = jnp.exp(lt - m1)
    weight = e / jnp.sum(e, axis=0, keepdims=True)

    eidx = jax.lax.broadcasted_iota(jnp.int32, lt.shape, 0)
    big = jnp.int32(N_EXPERTS)
    i1 = jnp.min(jnp.where(lt == m1, eidx, big), axis=0, keepdims=True)
    neg = jnp.float32(-jnp.inf)
    rest = jnp.where(eidx == i1, neg, lt)
    m2 = jnp.max(rest, axis=0, keepdims=True)
    i2 = jnp.min(jnp.where(rest == m2, eidx, big), axis=0, keepdims=True)
    mask = (eidx == i1) | (eidx == i2)

    mask_ref[...] = mask.astype(jnp.float32)
    weight_ref[...] = weight


@jax.jit
def kernel(h_td, W):
    wt = W.T
    grid = (T // BLOCK_T,)
    out_shape = (
        jax.ShapeDtypeStruct((N_EXPERTS, T), jnp.float32),
        jax.ShapeDtypeStruct((N_EXPERTS, T), jnp.float32),
        jax.ShapeDtypeStruct((N_EXPERTS, T), jnp.float32),
    )
    out_spec = pl.BlockSpec((N_EXPERTS, BLOCK_T), lambda i: (0, i))
    mask_f, weight, logits = pl.pallas_call(
        _router_kernel,
        grid=grid,
        in_specs=[
            pl.BlockSpec((BLOCK_T, D_CHUNK), lambda i: (i, 0)),
            pl.BlockSpec((BLOCK_T, D_CHUNK), lambda i: (i, 1)),
            pl.BlockSpec((BLOCK_T, D_CHUNK), lambda i: (i, 2)),
            pl.BlockSpec((D_MODEL, N_EXPERTS), lambda i: (0, 0)),
        ],
        out_specs=(out_spec, out_spec, out_spec),
        out_shape=out_shape,
    )(h_td, h_td, h_td, wt)
    return (mask_f.T.astype(bool), weight.T, logits.T)


# speedup vs baseline: 1.0449x; 1.0449x over previous
import functools

import jax
import jax.numpy as jnp
from jax.experimental import pallas as pl
from jax.experimental.pallas import tpu as pltpu

T = 32768
D_MODEL = 768
N_EXPERTS = 8
TOP_K = 2

BLOCK_T = 4096


def _router_kernel(h_ref, wt_ref, mask_ref, weight_ref, logits_ref):
    x = h_ref[...]
    wt = wt_ref[...]
    logits = jax.lax.dot_general(
        x, wt, (((1,), (0,)), ((), ())), preferred_element_type=jnp.float32
    )

    lt = logits.T
    logits_ref[...] = lt

    m1 = jnp.max(lt, axis=0, keepdims=True)
    e = jnp.exp(lt - m1)
    weight = e / jnp.sum(e, axis=0, keepdims=True)

    eidx = jax.lax.broadcasted_iota(jnp.int32, lt.shape, 0)
    big = jnp.int32(N_EXPERTS)
    i1 = jnp.min(jnp.where(lt == m1, eidx, big), axis=0, keepdims=True)
    neg = jnp.float32(-jnp.inf)
    rest = jnp.where(eidx == i1, neg, lt)
    m2 = jnp.max(rest, axis=0, keepdims=True)
    i2 = jnp.min(jnp.where(rest == m2, eidx, big), axis=0, keepdims=True)
    mask = (eidx == i1) | (eidx == i2)

    mask_ref[...] = mask.astype(jnp.float32)
    weight_ref[...] = weight


@jax.jit
def kernel(h_td, W):
    wt = W.T
    grid = (T // BLOCK_T,)
    out_shape = (
        jax.ShapeDtypeStruct((N_EXPERTS, T), jnp.float32),
        jax.ShapeDtypeStruct((N_EXPERTS, T), jnp.float32),
        jax.ShapeDtypeStruct((N_EXPERTS, T), jnp.float32),
    )
    out_spec = pl.BlockSpec((N_EXPERTS, BLOCK_T), lambda i: (0, i))
    mask_f, weight, logits = pl.pallas_call(
        _router_kernel,
        grid=grid,
        in_specs=[
            pl.BlockSpec((BLOCK_T, D_MODEL), lambda i: (i, 0)),
            pl.BlockSpec((D_MODEL, N_EXPERTS), lambda i: (0, 0)),
        ],
        out_specs=(out_spec, out_spec, out_spec),
        out_shape=out_shape,
    )(h_td, wt)
    return (mask_f.T.astype(bool), weight.T, logits.T)
